# baseline (speedup 1.0000x reference)
import functools

import jax
import jax.numpy as jnp
from jax import lax
from jax.experimental import pallas as pl
from jax.experimental.pallas import tpu as pltpu
from jax.experimental.pallas import tpu_sc as plsc

VOCAB_SIZE = 1000000
TIME_DIM = 32
BATCH = 4096
SEQ_LEN = 50

_NW = 32
_BB = BATCH // _NW
_TBLK = 2048
_NBLK = 123
_STRIP = _TBLK * _NBLK
_ROWS_OUT = BATCH * SEQ_LEN // 4


def _tc_relayout_body(x0_ref, x1_ref, x2_ref, x3_ref, rows_ref):
    for q, ref in enumerate((x0_ref, x1_ref, x2_ref, x3_ref)):
        rows_ref[:, pl.ds(q * TIME_DIM, TIME_DIM)] = ref[...].T


_tc_relayout = pl.pallas_call(
    _tc_relayout_body,
    grid=(_NBLK,),
    in_specs=[
        pl.BlockSpec((TIME_DIM, _TBLK), functools.partial(
            lambda q, i: (0, jnp.minimum(q * _NBLK + i,
                                         VOCAB_SIZE // _TBLK)), q))
        for q in range(4)
    ],
    out_specs=pl.BlockSpec((_TBLK, 128), lambda i: (i, 0)),
    out_shape=jax.ShapeDtypeStruct((_STRIP, 128), jnp.float32),
)


def _tc_out_transpose_body(rows_ref, out_ref):
    x = rows_ref[...]
    for q in range(4):
        out_ref[0, :, pl.ds(q * TIME_DIM, TIME_DIM)] = (
            x[:, q * TIME_DIM:(q + 1) * TIME_DIM].T)


_tc_out_transpose = pl.pallas_call(
    _tc_out_transpose_body,
    grid=(SEQ_LEN, _NW),
    in_specs=[pl.BlockSpec((TIME_DIM, 128), lambda s, w: (s * _NW + w, 0))],
    out_specs=pl.BlockSpec((1, TIME_DIM, _BB), lambda s, w: (s, 0, w)),
    out_shape=jax.ShapeDtypeStruct((SEQ_LEN, TIME_DIM, BATCH), jnp.float32),
)


@functools.partial(
    pl.kernel,
    mesh=plsc.VectorSubcoreMesh(core_axis_name="c", subcore_axis_name="s"),
    out_type=jax.ShapeDtypeStruct((_ROWS_OUT, 128), jnp.float32),
    scratch_types=[
        pltpu.VMEM((SEQ_LEN, _BB), jnp.int32),
        pltpu.VMEM((2, _BB), jnp.int32),
        pltpu.VMEM((2, _BB), jnp.int32),
        pltpu.VMEM((2, _BB, 128), jnp.float32),
        pltpu.VMEM((2, TIME_DIM, 128), jnp.float32),
        pltpu.SemaphoreType.DMA,
        pltpu.SemaphoreType.DMA,
    ],
    compiler_params=pltpu.CompilerParams(
        use_tc_tiling_on_sc=True, needs_layout_passes=False),
)
def _sc_gather(rows_tab, t_ids, rows_out, idx_v, ridx_v, coff_v, rows_v,
               orow_v, sem0, sem1):
    w = lax.axis_index("s") * 2 + lax.axis_index("c")
    sems = (sem0, sem1)
    pltpu.sync_copy(t_ids.at[:, pl.ds(w * _BB, _BB)], idx_v)

    def prep_and_fire(s, p):
        for k in range(_BB // 16):
            v = idx_v[s, pl.ds(k * 16, 16)]
            ridx_v[p, pl.ds(k * 16, 16)] = lax.rem(v, _STRIP)
            coff_v[p, pl.ds(k * 16, 16)] = lax.shift_left(
                lax.div(v, _STRIP), 5)
        pltpu.async_copy(rows_tab.at[ridx_v.at[p]], rows_v.at[p], sems[p])

    def drain_extract_write(s, p):
        pltpu.make_async_copy(rows_tab.at[ridx_v.at[p]], rows_v.at[p],
                              sems[p]).wait()
        for k in range(_BB // 16):
            offv = coff_v[p, pl.ds(k * 16, 16)]
            for j in range(16):
                t = k * 16 + j
                off = offv[j]
                u, q = t % TIME_DIM, t // TIME_DIM
                orow_v[p, u, pl.ds(q * TIME_DIM, 16)] = (
                    rows_v[p, t, pl.ds(off, 16)])
                orow_v[p, u, pl.ds(q * TIME_DIM + 16, 16)] = (
                    rows_v[p, t, pl.ds(off + 16, 16)])
        pltpu.sync_copy(
            orow_v.at[p],
            rows_out.at[pl.ds(s * (BATCH // 4) + w * TIME_DIM, TIME_DIM), :])

    prep_and_fire(0, 0)

    def body(i, carry):
        for j in range(2):
            s = 2 * i + j
            @pl.when(s + 1 < SEQ_LEN)
            def _():
                prep_and_fire(s + 1, (j + 1) % 2)
            drain_extract_write(s, j)
        return carry

    lax.fori_loop(0, SEQ_LEN // 2, body, 0)


def kernel(token_ids, time_embeddings):
    t_tab = time_embeddings.T
    rows_tab = _tc_relayout(t_tab, t_tab, t_tab, t_tab)
    rows_seq = _sc_gather(rows_tab, token_ids.T.astype(jnp.int32))
    out3 = _tc_out_transpose(rows_seq)
    return out3.transpose(2, 0, 1)


# speedup vs baseline: 2.8989x; 2.8989x over previous
import functools

import jax
import jax.numpy as jnp
from jax import lax
from jax.experimental import pallas as pl
from jax.experimental.pallas import tpu as pltpu
from jax.experimental.pallas import tpu_sc as plsc

VOCAB_SIZE = 1000000
TIME_DIM = 32
BATCH = 4096
SEQ_LEN = 50

_NW = 32
_BB = BATCH // _NW
_TBLK = 2048
_NBLK = 123
_STRIP = _TBLK * _NBLK
_ROWS_OUT = BATCH * SEQ_LEN // 4

_DOT_DIMS = (((0,), (0,)), ((), ()))
_DOT_DIMS_T = (((1,), (1,)), ((), ()))


def _tc_relayout_body(x0_ref, x1_ref, x2_ref, x3_ref, rows_ref):
    eye = jnp.eye(TIME_DIM, dtype=jnp.float32)
    for q, ref in enumerate((x0_ref, x1_ref, x2_ref, x3_ref)):
        rows_ref[:, pl.ds(q * TIME_DIM, TIME_DIM)] = lax.dot_general(
            ref[...], eye, _DOT_DIMS, preferred_element_type=jnp.float32)


_tc_relayout = pl.pallas_call(
    _tc_relayout_body,
    grid=(_NBLK,),
    in_specs=[
        pl.BlockSpec((TIME_DIM, _TBLK), functools.partial(
            lambda q, i: (0, jnp.minimum(q * _NBLK + i,
                                         VOCAB_SIZE // _TBLK)), q))
        for q in range(4)
    ],
    out_specs=pl.BlockSpec((_TBLK, 128), lambda i: (i, 0)),
    out_shape=jax.ShapeDtypeStruct((_STRIP, 128), jnp.float32),
)


def _tc_out_transpose_body(rows_ref, out_ref):
    eye = jnp.eye(TIME_DIM, dtype=jnp.float32)
    x = rows_ref[...]
    for q in range(4):
        piece = x[:, q * TIME_DIM:(q + 1) * TIME_DIM]
        out_ref[0, :, pl.ds(q * 1024, 1024)] = lax.dot_general(
            eye, piece, _DOT_DIMS_T, preferred_element_type=jnp.float32)


_tc_out_transpose = pl.pallas_call(
    _tc_out_transpose_body,
    grid=(SEQ_LEN,),
    in_specs=[pl.BlockSpec((1024, 128), lambda s: (s, 0))],
    out_specs=pl.BlockSpec((1, TIME_DIM, BATCH), lambda s: (s, 0, 0)),
    out_shape=jax.ShapeDtypeStruct((SEQ_LEN, TIME_DIM, BATCH), jnp.float32),
)


@functools.partial(
    pl.kernel,
    mesh=plsc.VectorSubcoreMesh(core_axis_name="c", subcore_axis_name="s"),
    out_type=jax.ShapeDtypeStruct((_ROWS_OUT, 128), jnp.float32),
    scratch_types=[
        pltpu.VMEM((4 * SEQ_LEN, 128), jnp.int32),
        pltpu.VMEM((2, _BB), jnp.int32),
        pltpu.VMEM((2, _BB), jnp.int32),
        pltpu.VMEM((2, _BB, 128), jnp.float32),
        pltpu.VMEM((2, TIME_DIM, 128), jnp.float32),
        pltpu.SemaphoreType.DMA,
        pltpu.SemaphoreType.DMA,
    ],
    compiler_params=pltpu.CompilerParams(
        use_tc_tiling_on_sc=True, needs_layout_passes=False),
)
def _sc_gather(rows_tab, t_ids, rows_out, idx_v, ridx_v, coff_v, rows_v,
               orow_v, sem0, sem1):
    w = lax.axis_index("s") * 2 + lax.axis_index("c")
    sems = (sem0, sem1)
    for q in range(4):
        pltpu.sync_copy(
            t_ids.at[:, pl.ds(q * 1024 + (w // 4) * 128, 128)],
            idx_v.at[pl.ds(q * SEQ_LEN, SEQ_LEN), :])
    cbase = (w % 4) * TIME_DIM

    def prep_and_fire(s, p):
        for q in range(4):
            for c in range(2):
                v = idx_v[q * SEQ_LEN + s, pl.ds(cbase + c * 16, 16)]
                dst = pl.ds(q * TIME_DIM + c * 16, 16)
                ridx_v[p, dst] = lax.rem(v, _STRIP)
                coff_v[p, dst] = lax.shift_left(lax.div(v, _STRIP), 5)
        pltpu.async_copy(rows_tab.at[ridx_v.at[p]], rows_v.at[p], sems[p])

    def drain_extract_write(s, p):
        pltpu.make_async_copy(rows_tab.at[ridx_v.at[p]], rows_v.at[p],
                              sems[p]).wait()
        for q in range(4):
            for c in range(2):
                offv = coff_v[p, pl.ds(q * TIME_DIM + c * 16, 16)]
                for j in range(16):
                    u = c * 16 + j
                    t = q * TIME_DIM + u
                    off = offv[j]
                    orow_v[p, u, pl.ds(q * TIME_DIM, 16)] = (
                        rows_v[p, t, pl.ds(off, 16)])
                    orow_v[p, u, pl.ds(q * TIME_DIM + 16, 16)] = (
                        rows_v[p, t, pl.ds(off + 16, 16)])
        pltpu.sync_copy(
            orow_v.at[p],
            rows_out.at[pl.ds(s * (BATCH // 4) + w * TIME_DIM, TIME_DIM), :])

    prep_and_fire(0, 0)

    def body(i, carry):
        for j in range(2):
            s = 2 * i + j
            @pl.when(s + 1 < SEQ_LEN)
            def _():
                prep_and_fire(s + 1, (j + 1) % 2)
            drain_extract_write(s, j)
        return carry

    lax.fori_loop(0, SEQ_LEN // 2, body, 0)


def kernel(token_ids, time_embeddings):
    t_tab = time_embeddings.T
    rows_tab = _tc_relayout(t_tab, t_tab, t_tab, t_tab)
    rows_seq = _sc_gather(rows_tab, token_ids.T.astype(jnp.int32))
    out3 = _tc_out_transpose(rows_seq)
    return out3.transpose(2, 0, 1)
